# baseline (speedup 1.0000x reference)
import jax
import jax.numpy as jnp
from jax.experimental import pallas as pl
from jax.experimental.pallas import tpu as pltpu

_CHUNK = 5000
_NBUF = 4


def _mk_kernel(n, in_ch, out_ch):
    nstep = n // _CHUNK

    def body(x_hbm, w_ref, o_hbm, xbuf, obuf, insem, outsem):
        def in_copy(i):
            return pltpu.make_async_copy(
                x_hbm.at[pl.ds(i * _CHUNK, _CHUNK), :],
                xbuf.at[i % _NBUF],
                insem.at[i % _NBUF])

        def out_copy(i):
            return pltpu.make_async_copy(
                obuf.at[i % _NBUF],
                o_hbm.at[pl.ds(i * _CHUNK, _CHUNK), :],
                outsem.at[i % _NBUF])

        for s in range(min(_NBUF, nstep)):
            in_copy(s).start()
        for i in range(nstep):
            slot = i % _NBUF
            in_copy(i).wait()
            if i >= _NBUF:
                out_copy(i - _NBUF).wait()
            obuf[slot] = jnp.dot(xbuf[slot], w_ref[...],
                                 preferred_element_type=jnp.float32)
            out_copy(i).start()
            nxt = i + _NBUF
            if nxt < nstep:
                in_copy(nxt).start()
        for i in range(max(0, nstep - _NBUF), nstep):
            out_copy(i).wait()

    return body


def kernel(x_src, W):
    n, in_ch = x_src.shape
    out_ch = W.shape[0]
    wt = W.T
    return pl.pallas_call(
        _mk_kernel(n, in_ch, out_ch),
        in_specs=[
            pl.BlockSpec(memory_space=pl.ANY),
            pl.BlockSpec((in_ch, out_ch), lambda: (0, 0)),
        ],
        out_specs=pl.BlockSpec(memory_space=pl.ANY),
        out_shape=jax.ShapeDtypeStruct((n, out_ch), jnp.float32),
        scratch_shapes=[
            pltpu.VMEM((_NBUF, _CHUNK, in_ch), jnp.float32),
            pltpu.VMEM((_NBUF, _CHUNK, out_ch), jnp.float32),
            pltpu.SemaphoreType.DMA((_NBUF,)),
            pltpu.SemaphoreType.DMA((_NBUF,)),
        ],
    )(x_src, wt)


# speedup vs baseline: 1.1070x; 1.1070x over previous
import jax
import jax.numpy as jnp
from jax.experimental import pallas as pl
from jax.experimental.pallas import tpu as pltpu

_BLK = 20000


def _mm_kernel(x_ref, w_ref, o_ref):
    o_ref[...] = jax.lax.dot_general(
        x_ref[...], w_ref[...],
        dimension_numbers=(((1,), (1,)), ((), ())),
        preferred_element_type=jnp.float32)


def kernel(x_src, W):
    n, in_ch = x_src.shape
    out_ch = W.shape[0]
    return pl.pallas_call(
        _mm_kernel,
        grid=(n // _BLK,),
        in_specs=[
            pl.BlockSpec((_BLK, in_ch), lambda i: (i, 0)),
            pl.BlockSpec((out_ch, in_ch), lambda i: (0, 0)),
        ],
        out_specs=pl.BlockSpec((_BLK, out_ch), lambda i: (i, 0)),
        out_shape=jax.ShapeDtypeStruct((n, out_ch), jnp.float32),
        compiler_params=pltpu.CompilerParams(
            dimension_semantics=("parallel",),
        ),
    )(x_src, W)
